# Initial kernel scaffold; baseline (speedup 1.0000x reference)
import jax
import jax.numpy as jnp
from jax.experimental import pallas as pl


def kernel(pos_u, pos_w, neg_w, u_weight, w_weight):
    raise NotImplementedError("write your pallas kernel here")


# speedup vs baseline: 2.4235x; 2.4235x over previous
import functools

import jax
import jax.numpy as jnp
from jax import lax
from jax.experimental import pallas as pl
from jax.experimental.pallas import tpu as pltpu
from jax.experimental.pallas import tpu_sc as plsc

VOCAB, D, B, C, K = 100000, 128, 16384, 10, 20
NC, NS = 2, 16
NW = NC * NS
EPW = B // NW
EC = 64
NCHUNK = EPW // EC
ROWS_U = EC * C
NG = ROWS_U // 128
NNEG = EPW * K // 128
NPW = EPW // 128


def _make_sc_gather():
    mesh = plsc.VectorSubcoreMesh(core_axis_name="c", subcore_axis_name="s",
                                  num_cores=NC, num_subcores=NS)

    @functools.partial(
        pl.kernel,
        out_type=[
            jax.ShapeDtypeStruct((B, D), jnp.float32),
            jax.ShapeDtypeStruct((B, D), jnp.float32),
            jax.ShapeDtypeStruct((B * K, D), jnp.float32),
        ],
        mesh=mesh,
        scratch_types=[
            pltpu.VMEM((EPW * C // 128, 128), jnp.int32),
            pltpu.VMEM((NNEG, 128), jnp.int32),
            pltpu.VMEM((8, 128), jnp.int32),
            pltpu.VMEM((ROWS_U, D), jnp.float32),
            pltpu.VMEM((EC, D), jnp.float32),
            pltpu.VMEM((128, D), jnp.float32),
            pltpu.SemaphoreType.DMA,
        ],
    )
    def sc_gather(posu2d, posw2d, neg2d, uw, ww, h_out, pw_out, nw_out,
                  uidx, negidx, pwidx, urows, hbuf, wrows, sem):
        wid = lax.axis_index("s") * NC + lax.axis_index("c")

        pltpu.sync_copy(posu2d.at[pl.ds(wid * (EPW * C // 128),
                                        EPW * C // 128)], uidx)
        pltpu.sync_copy(neg2d.at[pl.ds(wid * NNEG, NNEG)], negidx)
        pltpu.sync_copy(posw2d.at[pl.ds((wid // 2) * 8, 8)], pwidx)

        def ctx_chunk(i, carry):
            cps = [pltpu.async_copy(uw.at[uidx.at[i * NG + g]],
                                    urows.at[pl.ds(g * 128, 128)], sem)
                   for g in range(NG)]
            for cp in cps:
                cp.wait()

            def ex(e, c2):
                r0 = e * C
                for d in range(D // 16):
                    sl = pl.ds(d * 16, 16)
                    acc = urows[r0, sl]
                    for cc in range(1, C):
                        acc = acc + urows[r0 + cc, sl]
                    hbuf[e, sl] = acc
                return c2
            lax.fori_loop(0, EC, ex, 0)
            pltpu.sync_copy(hbuf, h_out.at[pl.ds(wid * EPW + i * EC, EC)])
            return carry
        lax.fori_loop(0, NCHUNK, ctx_chunk, 0)

        def neg_chunk(j, carry):
            pltpu.async_copy(ww.at[negidx.at[j]], wrows, sem).wait()
            pltpu.sync_copy(wrows,
                            nw_out.at[pl.ds(wid * EPW * K + j * 128, 128)])
            return carry
        lax.fori_loop(0, NNEG, neg_chunk, 0)

        def pw_chunk(j, carry):
            pltpu.async_copy(ww.at[pwidx.at[(wid % 2) * NPW + j]],
                             wrows, sem).wait()
            pltpu.sync_copy(wrows,
                            pw_out.at[pl.ds(wid * EPW + j * 128, 128)])
            return carry
        lax.fori_loop(0, NPW, pw_chunk, 0)

    return sc_gather


_BLK = 1024


def _tc_loss_body(h_ref, pw_ref, nw_ref, out_ref):
    h = h_ref[...]
    s2 = jnp.sum(h * pw_ref[...], axis=1)
    ns = jnp.sum(nw_ref[...] * h[:, None, :], axis=2)
    part = jnp.sum(jax.nn.log_sigmoid(s2)) + jnp.sum(jax.nn.log_sigmoid(-ns))

    @pl.when(pl.program_id(0) == 0)
    def _():
        out_ref[0, 0] = 0.0

    out_ref[0, 0] += part


_tc_loss = pl.pallas_call(
    _tc_loss_body,
    grid=(B // _BLK,),
    in_specs=[
        pl.BlockSpec((_BLK, D), lambda i: (i, 0)),
        pl.BlockSpec((_BLK, D), lambda i: (i, 0)),
        pl.BlockSpec((_BLK, K, D), lambda i: (i, 0, 0)),
    ],
    out_specs=pl.BlockSpec((1, 1), lambda i: (0, 0), memory_space=pltpu.SMEM),
    out_shape=jax.ShapeDtypeStruct((1, 1), jnp.float32),
)


def kernel(pos_u, pos_w, neg_w, u_weight, w_weight):
    posu2d = pos_u.reshape(B * C // 128, 128)
    posw2d = pos_w.reshape(B // 128, 128)
    neg2d = neg_w.reshape(B * K // 128, 128)
    h, pw, nw = _make_sc_gather()(posu2d, posw2d, neg2d, u_weight, w_weight)
    nw3 = nw.reshape(B, K, D)
    loss = _tc_loss(h, pw, nw3)
    return -loss[0, 0]


# speedup vs baseline: 3.1507x; 1.3001x over previous
import functools

import jax
import jax.numpy as jnp
from jax import lax
from jax.experimental import pallas as pl
from jax.experimental.pallas import tpu as pltpu
from jax.experimental.pallas import tpu_sc as plsc

VOCAB, D, B, C, K = 100000, 128, 16384, 10, 20
NC, NS = 2, 16
NW = NC * NS
EPW = B // NW
EC = 64
NCHUNK = EPW // EC
ROWS_U = EC * C
NG = ROWS_U // 128
NNEG = EPW * K // 128
NPW = EPW // 128
NDV = D // 16


def _row_slices(ref, r):
    return [ref[r, pl.ds(d * 16, 16)] for d in range(NDV)]


def _dot_partial(hv, ref, r):
    p = hv[0] * ref[r, pl.ds(0, 16)]
    for d in range(1, NDV):
        p = p + hv[d] * ref[r, pl.ds(d * 16, 16)]
    return p


def _make_sc_fused():
    mesh = plsc.VectorSubcoreMesh(core_axis_name="c", subcore_axis_name="s",
                                  num_cores=NC, num_subcores=NS)

    @functools.partial(
        pl.kernel,
        out_type=[
            jax.ShapeDtypeStruct((B,), jnp.float32),
            jax.ShapeDtypeStruct((B * K,), jnp.float32),
            jax.ShapeDtypeStruct((B, D), jnp.float32),
        ],
        mesh=mesh,
        compiler_params=pltpu.CompilerParams(needs_layout_passes=False),
        scratch_types=[
            pltpu.VMEM((EPW * C // 128, 128), jnp.int32),
            pltpu.VMEM((NNEG, 128), jnp.int32),
            pltpu.VMEM((8, 128), jnp.int32),
            pltpu.VMEM((ROWS_U, D), jnp.float32),
            pltpu.VMEM((EC, D), jnp.float32),
            pltpu.VMEM((128, D), jnp.float32),
            pltpu.VMEM((16, 16), jnp.float32),
            pltpu.VMEM((128,), jnp.float32),
            pltpu.VMEM((EC * K,), jnp.float32),
            pltpu.SemaphoreType.DMA,
        ],
    )
    def sc_fused(posu2d, posw2d, neg2d, uw, ww, s2_out, ns_out, h_out,
                 uidx, negidx, pwidx, rows, hbuf, wrows, part, s2buf, nsbuf,
                 sem):
        wid = lax.axis_index("s") * NC + lax.axis_index("c")
        rowidx = jnp.arange(16, dtype=jnp.int32)

        def flush16(dst_ref, dst_off):
            acc = plsc.load_gather(part, [rowidx,
                                          jnp.zeros(16, jnp.int32)])
            for j in range(1, 16):
                acc = acc + plsc.load_gather(
                    part, [rowidx, jnp.full(16, j, jnp.int32)])
            dst_ref[pl.ds(dst_off, 16)] = acc

        pltpu.sync_copy(posu2d.at[pl.ds(wid * (EPW * C // 128),
                                        EPW * C // 128)], uidx)
        pltpu.sync_copy(neg2d.at[pl.ds(wid * NNEG, NNEG)], negidx)
        pltpu.sync_copy(posw2d.at[pl.ds((wid // 2) * 8, 8)], pwidx)

        def chunk(i, carry):
            cps = [pltpu.async_copy(uw.at[uidx.at[i * NG + g]],
                                    rows.at[pl.ds(g * 128, 128)], sem)
                   for g in range(NG)]
            for cp in cps:
                cp.wait()

            def ex(e, c2):
                r0 = e * C
                for d in range(NDV):
                    sl = pl.ds(d * 16, 16)
                    acc = rows[r0, sl]
                    for cc in range(1, C):
                        acc = acc + rows[r0 + cc, sl]
                    hbuf[e, sl] = acc
                return c2
            lax.fori_loop(0, EC, ex, 0)
            pltpu.sync_copy(hbuf, h_out.at[pl.ds(wid * EPW + i * EC, EC)])

            def half(h2, c2):
                cps2 = [pltpu.async_copy(
                            ww.at[negidx.at[i * 2 * NG + h2 * NG + g]],
                            rows.at[pl.ds(g * 128, 128)], sem)
                        for g in range(NG)]
                for cp in cps2:
                    cp.wait()

                def grp(g, c3):
                    for q in range(4):
                        lb = g * 4 + q
                        hv = _row_slices(hbuf, h2 * 32 + lb)
                        for k in range(K):
                            t = q * K + k
                            part[t % 16, :] = (
                                _dot_partial(hv, rows, lb * K + k))
                            if t % 16 == 15:
                                flush16(nsbuf, h2 * 640 + g * 80
                                        + (t // 16) * 16)
                    return c3
                lax.fori_loop(0, 8, grp, 0)
                return c2
            lax.fori_loop(0, 2, half, 0)

            pltpu.sync_copy(nsbuf,
                            ns_out.at[pl.ds(wid * EPW * K + i * EC * K,
                                            EC * K)])
            return carry
        lax.fori_loop(0, NCHUNK, chunk, 0)

        def pw_chunk(j, carry):
            pltpu.async_copy(ww.at[pwidx.at[(wid % 2) * NPW + j]],
                             wrows, sem).wait()
            pltpu.sync_copy(h_out.at[pl.ds(wid * EPW + j * 128, 128)],
                            rows.at[pl.ds(0, 128)])

            def grp(g, c2):
                for q in range(16):
                    le = g * 16 + q
                    hv = _row_slices(rows, le)
                    part[q, :] = _dot_partial(hv, wrows, le)
                flush16(s2buf, g * 16)
                return c2
            lax.fori_loop(0, 8, grp, 0)
            pltpu.sync_copy(s2buf.at[pl.ds(0, 128)],
                            s2_out.at[pl.ds(wid * EPW + j * 128, 128)])
            return carry
        lax.fori_loop(0, NPW, pw_chunk, 0)

    return sc_fused


def _tc_loss_body(s2_ref, ns_ref, out_ref):
    part = (jnp.sum(jax.nn.log_sigmoid(s2_ref[...]))
            + jnp.sum(jax.nn.log_sigmoid(-ns_ref[...])))
    out_ref[0, 0] = -part


_tc_loss = pl.pallas_call(
    _tc_loss_body,
    out_specs=pl.BlockSpec(memory_space=pltpu.SMEM),
    out_shape=jax.ShapeDtypeStruct((1, 1), jnp.float32),
)


def kernel(pos_u, pos_w, neg_w, u_weight, w_weight):
    posu2d = pos_u.reshape(B * C // 128, 128)
    posw2d = pos_w.reshape(B // 128, 128)
    neg2d = neg_w.reshape(B * K // 128, 128)
    s2, ns, _h = _make_sc_fused()(posu2d, posw2d, neg2d, u_weight, w_weight)
    loss = _tc_loss(s2.reshape(B // 128, 128), ns.reshape(B * K // 128, 128))
    return loss[0, 0]


# speedup vs baseline: 4.1971x; 1.3321x over previous
import functools

import jax
import jax.numpy as jnp
from jax import lax
from jax.experimental import pallas as pl
from jax.experimental.pallas import tpu as pltpu
from jax.experimental.pallas import tpu_sc as plsc

VOCAB, D, B, C, K = 100000, 128, 16384, 10, 20
NC, NS = 2, 16
NW = NC * NS
EPW = B // NW
EC = 64
NCHUNK = EPW // EC
ROWS_U = EC * C
NG = ROWS_U // 128
NB = EC * K // 128
NDV = D // 16


def _hrow(ref, r):
    return [ref[r, pl.ds(d * 16, 16)] for d in range(NDV)]


def _dot_partial(hv, ref, r):
    p = hv[0] * ref[r, pl.ds(0, 16)]
    for d in range(1, NDV):
        p = p + ref[r, pl.ds(d * 16, 16)] * hv[d]
    return p


def _make_sc_fused():
    mesh = plsc.VectorSubcoreMesh(core_axis_name="c", subcore_axis_name="s",
                                  num_cores=NC, num_subcores=NS)

    @functools.partial(
        pl.kernel,
        out_type=[
            jax.ShapeDtypeStruct((B,), jnp.float32),
            jax.ShapeDtypeStruct((B * K,), jnp.float32),
        ],
        mesh=mesh,
        compiler_params=pltpu.CompilerParams(needs_layout_passes=False),
        scratch_types=[
            pltpu.VMEM((EPW * C // 128, 128), jnp.int32),
            pltpu.VMEM((EPW * K // 128, 128), jnp.int32),
            pltpu.VMEM((8, 128), jnp.int32),
            pltpu.VMEM((ROWS_U, D), jnp.float32),
            pltpu.VMEM((EC, D), jnp.float32),
            pltpu.VMEM((128, D), jnp.float32),
            pltpu.VMEM((16, 16), jnp.float32),
            pltpu.VMEM((EC,), jnp.float32),
            pltpu.VMEM((EC * K,), jnp.float32),
            pltpu.SemaphoreType.DMA,
            pltpu.SemaphoreType.DMA,
            pltpu.SemaphoreType.DMA,
            pltpu.SemaphoreType.DMA,
        ],
    )
    def sc_fused(posu2d, posw2d, neg2d, uw, ww, s2_out, ns_out,
                 uidx, negidx, pwidx, rows, hbuf, pwrows, part, s2buf, nsbuf,
                 usem, nsem0, nsem1, pwsem):
        wid = lax.axis_index("s") * NC + lax.axis_index("c")
        rowidx = jnp.arange(16, dtype=jnp.int32)

        def flush16(dst_ref, dst_off):
            acc = plsc.load_gather(part, [rowidx, jnp.zeros(16, jnp.int32)])
            for j in range(1, 16):
                acc = acc + plsc.load_gather(
                    part, [rowidx, jnp.full(16, j, jnp.int32)])
            dst_ref[pl.ds(dst_off, 16)] = acc

        pltpu.sync_copy(posu2d.at[pl.ds(wid * (EPW * C // 128),
                                        EPW * C // 128)], uidx)
        pltpu.sync_copy(neg2d.at[pl.ds(wid * (EPW * K // 128),
                                       EPW * K // 128)], negidx)
        pltpu.sync_copy(posw2d.at[pl.ds((wid // 2) * 8, 8)], pwidx)

        def neg_fire(i, b, slot):
            sem = nsem0 if slot == 0 else nsem1
            return pltpu.async_copy(ww.at[negidx.at[i * NB + b]],
                                    rows.at[pl.ds(slot * 128, 128)], sem)

        def chunk(i, carry):
            ucps = [pltpu.async_copy(uw.at[uidx.at[i * NG + g]],
                                     rows.at[pl.ds(g * 128, 128)], usem)
                    for g in range(NG)]
            @pl.when(i % 2 == 0)
            def _():
                pltpu.async_copy(ww.at[pwidx.at[(wid % 2) * (NCHUNK // 2)
                                                + i // 2]], pwrows, pwsem)
            for cp in ucps:
                cp.wait()

            def ex(e, c2):
                r0 = e * C
                for d in range(NDV):
                    sl = pl.ds(d * 16, 16)
                    acc = rows[r0, sl]
                    for cc in range(1, C):
                        acc = acc + rows[r0 + cc, sl]
                    hbuf[e, sl] = acc
                return c2
            lax.fori_loop(0, EC, ex, 0)

            @pl.when(i % 2 == 0)
            def _():
                pltpu.make_async_copy(
                    ww.at[pwidx.at[(wid % 2) * (NCHUNK // 2) + i // 2]],
                    pwrows, pwsem).wait()

            def pgrp(g, c2):
                for q in range(16):
                    le = g * 16 + q
                    hv = _hrow(hbuf, le)
                    part[q, :] = _dot_partial(hv, pwrows,
                                              (i % 2) * EC + le)
                flush16(s2buf, g * 16)
                return c2
            lax.fori_loop(0, EC // 16, pgrp, 0)
            pltpu.sync_copy(s2buf, s2_out.at[pl.ds(wid * EPW + i * EC, EC)])

            @pl.when(True)
            def _():
                neg_fire(i, 0, 0)

            def nbatch(b, c2):
                slotbase = (b % 2) * 128

                @pl.when((b % 2 == 0) & (b < NB - 1))
                def _():
                    neg_fire(i, b + 1, 1)

                @pl.when((b % 2 == 1) & (b < NB - 1))
                def _():
                    neg_fire(i, b + 1, 0)

                @pl.when(b % 2 == 0)
                def _():
                    pltpu.make_async_copy(ww.at[negidx.at[i * NB + b]],
                                          rows.at[pl.ds(0, 128)],
                                          nsem0).wait()

                @pl.when(b % 2 == 1)
                def _():
                    pltpu.make_async_copy(ww.at[negidx.at[i * NB + b]],
                                          rows.at[pl.ds(128, 128)],
                                          nsem1).wait()

                def ngrp(f, c3):
                    for q in range(16):
                        j = f * 16 + q
                        le = (b * 128 + j) // K
                        hv = _hrow(hbuf, le)
                        part[q, :] = _dot_partial(hv, rows, slotbase + j)
                    flush16(nsbuf, b * 128 + f * 16)
                    return c3
                lax.fori_loop(0, 8, ngrp, 0)
                return c2
            lax.fori_loop(0, NB, nbatch, 0)

            pltpu.sync_copy(nsbuf,
                            ns_out.at[pl.ds(wid * EPW * K + i * EC * K,
                                            EC * K)])
            return carry
        lax.fori_loop(0, NCHUNK, chunk, 0)

    return sc_fused


def _tc_loss_body(s2_ref, ns_ref, out_ref):
    part = (jnp.sum(jax.nn.log_sigmoid(s2_ref[...]))
            + jnp.sum(jax.nn.log_sigmoid(-ns_ref[...])))
    out_ref[0, 0] = -part


_tc_loss = pl.pallas_call(
    _tc_loss_body,
    out_specs=pl.BlockSpec(memory_space=pltpu.SMEM),
    out_shape=jax.ShapeDtypeStruct((1, 1), jnp.float32),
)


def kernel(pos_u, pos_w, neg_w, u_weight, w_weight):
    posu2d = pos_u.reshape(B * C // 128, 128)
    posw2d = pos_w.reshape(B // 128, 128)
    neg2d = neg_w.reshape(B * K // 128, 128)
    s2, ns = _make_sc_fused()(posu2d, posw2d, neg2d, u_weight, w_weight)
    loss = _tc_loss(s2.reshape(B // 128, 128), ns.reshape(B * K // 128, 128))
    return loss[0, 0]
